# baseline (speedup 1.0000x reference)
import functools

import jax
import jax.numpy as jnp
from jax import lax
from jax.experimental import pallas as pl
from jax.experimental.pallas import tpu as pltpu
from jax.experimental.pallas import tpu_sc as plsc

N_NODES = 10000
N_EDGES = 320000
D = 128
DP = D // 2

NC = 2
NS = 16
L = 16
NW = NC * NS
PER_W = N_EDGES // NW
CHUNK = 128
N_FULL = PER_W // CHUNK
TAIL = PER_W - N_FULL * CHUNK
TAIL_OFF = N_FULL * CHUNK
KU = 8

RBLK = 80
NBLK = N_NODES // RBLK

_SC_PARAMS = pltpu.CompilerParams(
    needs_layout_passes=False, use_tc_tiling_on_sc=False)
_MESH = dict(core_axis_name="c", subcore_axis_name="s")


def _prep_body(emb_hbm, ds_hbm, a_hbm, b_hbm, xbuf, dsv, apk, bpk,
               insem, outsem):
    wid = lax.axis_index("s") * NC + lax.axis_index("c")
    pltpu.sync_copy(ds_hbm, dsv)
    lane = lax.iota(jnp.int32, L)
    nblk = (NBLK - wid + NW - 1) // NW
    MAXB = (NBLK + NW - 1) // NW

    for i in range(MAXB):
        @pl.when(i < nblk)
        def _():
            pltpu.async_copy(
                emb_hbm.at[pl.ds((wid + i * NW) * RBLK, RBLK)],
                xbuf.at[pl.ds(i * RBLK, RBLK)], insem)
    for i in range(MAXB):
        @pl.when(i < nblk)
        def _():
            pltpu.make_async_copy(
                emb_hbm.at[pl.ds(0, RBLK)],
                xbuf.at[pl.ds(i * RBLK, RBLK)], insem).wait()

    def block_body(i, carry):
        row0 = (wid + i * NW) * RBLK
        rbase = i * RBLK
        for g in range(RBLK // L):
            rows = jnp.full((L,), g * L, jnp.int32) + rbase + lane

            def nbody(kk, carry):
                col, s2 = carry
                for _ in range(KU):
                    v = plsc.load_gather(xbuf, [rows, col])
                    s2 = s2 + v * v
                    col = (col + 1) & (D - 1)
                return col, s2

            _, s2 = lax.fori_loop(0, D // KU, nbody,
                                  (lane, jnp.zeros((L,), jnp.float32)))

            yi = jnp.full((L,), 0x5F3759DF, jnp.int32) - lax.shift_right_arithmetic(
                plsc.bitcast(s2, jnp.int32), 1)
            y = plsc.bitcast(yi, jnp.float32)
            half_s2 = 0.5 * s2
            for _ in range(4):
                y = y * (1.5 - half_s2 * y * y)

            def pbody(pp, col):
                for _ in range(KU // 2):
                    ce = col * 2
                    co = ce + 1
                    e = plsc.load_gather(xbuf, [rows, ce]) * y
                    o = plsc.load_gather(xbuf, [rows, co]) * y
                    b_pk = plsc.bitcast(
                        plsc.pack(e, o, format=plsc.PackFormat.INTERLEAVED),
                        jnp.int32)
                    dse = plsc.load_gather(dsv, [ce])
                    dso = plsc.load_gather(dsv, [co])
                    a_pk = plsc.bitcast(
                        plsc.pack(e * dse, o * dso,
                                  format=plsc.PackFormat.INTERLEAVED),
                        jnp.int32)
                    plsc.store_scatter(bpk, [rows, col], b_pk)
                    plsc.store_scatter(apk, [rows, col], a_pk)
                    col = (col + 1) & (DP - 1)
                return col

            lax.fori_loop(0, DP // (KU // 2), pbody, lane)
        pltpu.async_copy(apk.at[pl.ds(rbase, RBLK)],
                         a_hbm.at[pl.ds(row0, RBLK)], outsem)
        pltpu.async_copy(bpk.at[pl.ds(rbase, RBLK)],
                         b_hbm.at[pl.ds(row0, RBLK)], outsem)
        return carry

    lax.fori_loop(0, nblk, block_body, 0)
    for i in range(MAXB):
        @pl.when(i < nblk)
        def _():
            pltpu.make_async_copy(apk.at[pl.ds(i * RBLK, RBLK)],
                                  a_hbm.at[pl.ds(0, RBLK)], outsem).wait()
            pltpu.make_async_copy(bpk.at[pl.ds(i * RBLK, RBLK)],
                                  b_hbm.at[pl.ds(0, RBLK)], outsem).wait()


@jax.jit
def _sc_prep(emb, ds):
    f = functools.partial(
        pl.kernel,
        mesh=plsc.VectorSubcoreMesh(**_MESH),
        compiler_params=_SC_PARAMS,
        out_type=[
            jax.ShapeDtypeStruct((N_NODES, DP), jnp.int32),
            jax.ShapeDtypeStruct((N_NODES, DP), jnp.int32),
        ],
        scratch_types=[
            pltpu.VMEM((4 * RBLK, D), jnp.float32),
            pltpu.VMEM((D,), jnp.float32),
            pltpu.VMEM((4 * RBLK, DP), jnp.int32),
            pltpu.VMEM((4 * RBLK, DP), jnp.int32),
            pltpu.SemaphoreType.DMA,
            pltpu.SemaphoreType.DMA,
        ],
    )(_prep_body)
    return f(emb, ds)


def _edge_dot_body(a_hbm, b_hbm, edge_hbm, out_hbm,
                   sidx, didx, arows0, brows0, arows1, brows1, z,
                   sem0, sem1):
    wid = lax.axis_index("s") * NC + lax.axis_index("c")
    base = wid * PER_W

    pltpu.sync_copy(edge_hbm.at[0, pl.ds(base, PER_W)], sidx)
    pltpu.sync_copy(edge_hbm.at[1, pl.ds(base, PER_W)], didx)

    bufs = ((arows0, brows0, sem0), (arows1, brows1, sem1))
    himask = jnp.full((L,), -65536, jnp.int32)

    def start(c, p):
        ar, br, sem = bufs[p]
        off = c * CHUNK
        pltpu.async_copy(a_hbm.at[sidx.at[pl.ds(off, CHUNK)]], ar, sem)
        pltpu.async_copy(b_hbm.at[didx.at[pl.ds(off, CHUNK)]], br, sem)

    def finish(c, p):
        ar, br, sem = bufs[p]
        pltpu.make_async_copy(a_hbm.at[pl.ds(0, CHUNK)], ar, sem).wait()
        pltpu.make_async_copy(b_hbm.at[pl.ds(0, CHUNK)], br, sem).wait()
        lane = lax.iota(jnp.int32, L)
        for g in range(CHUNK // L):
            rows = jnp.full((L,), g * L, jnp.int32) + lane

            def kbody(kk, carry):
                col, acc_e, acc_o = carry
                for _ in range(KU):
                    va = plsc.load_gather(ar, [rows, col])
                    vb = plsc.load_gather(br, [rows, col])
                    ae = plsc.bitcast(lax.shift_left(va, 16), jnp.float32)
                    be = plsc.bitcast(lax.shift_left(vb, 16), jnp.float32)
                    ao = plsc.bitcast(va & himask, jnp.float32)
                    bo = plsc.bitcast(vb & himask, jnp.float32)
                    acc_e = acc_e + ae * be
                    acc_o = acc_o + ao * bo
                    col = (col + 1) & (DP - 1)
                return col, acc_e, acc_o

            _, acc_e, acc_o = lax.fori_loop(
                0, DP // KU, kbody,
                (lane, jnp.zeros((L,), jnp.float32),
                 jnp.zeros((L,), jnp.float32)))
            z[pl.ds(c * CHUNK + g * L, L)] = acc_e + acc_o

    start(0, 0)

    def pair_body(i, carry):
        c = 2 * i
        start(c + 1, 1)
        finish(c, 0)
        start(c + 2, 0)
        finish(c + 1, 1)
        return carry

    lax.fori_loop(0, (N_FULL - 2) // 2, pair_body, 0)
    start(N_FULL - 1, 1)
    finish(N_FULL - 2, 0)

    art = arows0.at[pl.ds(0, TAIL)]
    brt = brows0.at[pl.ds(0, TAIL)]
    pltpu.async_copy(a_hbm.at[sidx.at[pl.ds(TAIL_OFF, TAIL)]], art, sem0)
    pltpu.async_copy(b_hbm.at[didx.at[pl.ds(TAIL_OFF, TAIL)]], brt, sem0)
    finish(N_FULL - 1, 1)
    pltpu.make_async_copy(a_hbm.at[pl.ds(0, TAIL)], art, sem0).wait()
    pltpu.make_async_copy(b_hbm.at[pl.ds(0, TAIL)], brt, sem0).wait()
    lane_t = lax.iota(jnp.int32, L)

    def tbody(kk, carry):
        col, acc_e, acc_o = carry
        for _ in range(KU):
            va = plsc.load_gather(arows0, [lane_t, col])
            vb = plsc.load_gather(brows0, [lane_t, col])
            ae = plsc.bitcast(lax.shift_left(va, 16), jnp.float32)
            be = plsc.bitcast(lax.shift_left(vb, 16), jnp.float32)
            ao = plsc.bitcast(va & himask, jnp.float32)
            bo = plsc.bitcast(vb & himask, jnp.float32)
            acc_e = acc_e + ae * be
            acc_o = acc_o + ao * bo
            col = (col + 1) & (DP - 1)
        return col, acc_e, acc_o

    _, acc_e, acc_o = lax.fori_loop(
        0, DP // KU, tbody,
        (lane_t, jnp.zeros((L,), jnp.float32), jnp.zeros((L,), jnp.float32)))
    z[pl.ds(TAIL_OFF, L)] = acc_e + acc_o
    pltpu.sync_copy(z, out_hbm.at[pl.ds(base, PER_W)])


@jax.jit
def _sc_edge_dot(a, b, edge):
    f = functools.partial(
        pl.kernel,
        mesh=plsc.VectorSubcoreMesh(**_MESH),
        compiler_params=_SC_PARAMS,
        out_type=jax.ShapeDtypeStruct((N_EDGES,), jnp.float32),
        scratch_types=[
            pltpu.VMEM((PER_W,), jnp.int32),
            pltpu.VMEM((PER_W,), jnp.int32),
            pltpu.VMEM((CHUNK, DP), jnp.int32),
            pltpu.VMEM((CHUNK, DP), jnp.int32),
            pltpu.VMEM((CHUNK, DP), jnp.int32),
            pltpu.VMEM((CHUNK, DP), jnp.int32),
            pltpu.VMEM((PER_W,), jnp.float32),
            pltpu.SemaphoreType.DMA,
            pltpu.SemaphoreType.DMA,
        ],
    )(_edge_dot_body)
    return f(a, b, edge)


def kernel(emb, edge_index, d, scale):
    ds = (d * scale).astype(jnp.float32)
    a_p, b_p = _sc_prep(emb, ds)
    z = _sc_edge_dot(a_p, b_p, edge_index.astype(jnp.int32))
    return z.reshape(N_EDGES, 1)


# speedup vs baseline: 1.1816x; 1.1816x over previous
import functools

import jax
import jax.numpy as jnp
from jax import lax
from jax.experimental import pallas as pl
from jax.experimental.pallas import tpu as pltpu
from jax.experimental.pallas import tpu_sc as plsc

N_NODES = 10000
N_EDGES = 320000
D = 128
DP = D // 2

NC = 2
NS = 16
L = 16
NW = NC * NS
PER_W = N_EDGES // NW
CHUNK = 128
N_FULL = PER_W // CHUNK
TAIL = PER_W - N_FULL * CHUNK
TAIL_OFF = N_FULL * CHUNK
KU = 4

RBLK = 80
NBLK = N_NODES // RBLK

_SC_PARAMS = pltpu.CompilerParams(
    needs_layout_passes=False, use_tc_tiling_on_sc=False)
_MESH = dict(core_axis_name="c", subcore_axis_name="s")


def _prep_body(emb_hbm, ds_hbm, a_hbm, b_hbm, xbuf, dsv, apk, bpk,
               insem, outsem):
    wid = lax.axis_index("s") * NC + lax.axis_index("c")
    pltpu.sync_copy(ds_hbm, dsv)
    lane = lax.iota(jnp.int32, L)
    nblk = (NBLK - wid + NW - 1) // NW
    MAXB = (NBLK + NW - 1) // NW

    for i in range(MAXB):
        @pl.when(i < nblk)
        def _():
            pltpu.async_copy(
                emb_hbm.at[pl.ds((wid + i * NW) * RBLK, RBLK)],
                xbuf.at[pl.ds(i * RBLK, RBLK)], insem)
    for i in range(MAXB):
        @pl.when(i < nblk)
        def _():
            pltpu.make_async_copy(
                emb_hbm.at[pl.ds(0, RBLK)],
                xbuf.at[pl.ds(i * RBLK, RBLK)], insem).wait()

    def block_body(i, carry):
        row0 = (wid + i * NW) * RBLK
        rbase = i * RBLK
        for g in range(RBLK // L):
            rows = jnp.full((L,), g * L, jnp.int32) + rbase + lane

            def nbody(kk, carry):
                col, s2 = carry
                for _ in range(KU):
                    v = plsc.load_gather(xbuf, [rows, col])
                    s2 = s2 + v * v
                    col = (col + 1) & (D - 1)
                return col, s2

            _, s2 = lax.fori_loop(0, D // KU, nbody,
                                  (lane, jnp.zeros((L,), jnp.float32)))

            yi = jnp.full((L,), 0x5F3759DF, jnp.int32) - lax.shift_right_arithmetic(
                plsc.bitcast(s2, jnp.int32), 1)
            y = plsc.bitcast(yi, jnp.float32)
            half_s2 = 0.5 * s2
            for _ in range(4):
                y = y * (1.5 - half_s2 * y * y)

            def pbody(pp, col):
                for _ in range(KU // 2):
                    ce = col * 2
                    co = ce + 1
                    e = plsc.load_gather(xbuf, [rows, ce]) * y
                    o = plsc.load_gather(xbuf, [rows, co]) * y
                    b_pk = plsc.bitcast(
                        plsc.pack(e, o, format=plsc.PackFormat.INTERLEAVED),
                        jnp.int32)
                    dse = plsc.load_gather(dsv, [ce])
                    dso = plsc.load_gather(dsv, [co])
                    a_pk = plsc.bitcast(
                        plsc.pack(e * dse, o * dso,
                                  format=plsc.PackFormat.INTERLEAVED),
                        jnp.int32)
                    plsc.store_scatter(bpk, [rows, col], b_pk)
                    plsc.store_scatter(apk, [rows, col], a_pk)
                    col = (col + 1) & (DP - 1)
                return col

            lax.fori_loop(0, DP // (KU // 2), pbody, lane)
        pltpu.async_copy(apk.at[pl.ds(rbase, RBLK)],
                         a_hbm.at[pl.ds(row0, RBLK)], outsem)
        pltpu.async_copy(bpk.at[pl.ds(rbase, RBLK)],
                         b_hbm.at[pl.ds(row0, RBLK)], outsem)
        return carry

    lax.fori_loop(0, nblk, block_body, 0)
    for i in range(MAXB):
        @pl.when(i < nblk)
        def _():
            pltpu.make_async_copy(apk.at[pl.ds(i * RBLK, RBLK)],
                                  a_hbm.at[pl.ds(0, RBLK)], outsem).wait()
            pltpu.make_async_copy(bpk.at[pl.ds(i * RBLK, RBLK)],
                                  b_hbm.at[pl.ds(0, RBLK)], outsem).wait()


@jax.jit
def _sc_prep(emb, ds):
    f = functools.partial(
        pl.kernel,
        mesh=plsc.VectorSubcoreMesh(**_MESH),
        compiler_params=_SC_PARAMS,
        out_type=[
            jax.ShapeDtypeStruct((N_NODES, DP), jnp.int32),
            jax.ShapeDtypeStruct((N_NODES, DP), jnp.int32),
        ],
        scratch_types=[
            pltpu.VMEM((4 * RBLK, D), jnp.float32),
            pltpu.VMEM((D,), jnp.float32),
            pltpu.VMEM((4 * RBLK, DP), jnp.int32),
            pltpu.VMEM((4 * RBLK, DP), jnp.int32),
            pltpu.SemaphoreType.DMA,
            pltpu.SemaphoreType.DMA,
        ],
    )(_prep_body)
    return f(emb, ds)


def _edge_dot_body(a_hbm, b_hbm, edge_hbm, out_hbm,
                   sidx, didx, arows0, brows0, arows1, brows1, z,
                   sem0, sem1):
    wid = lax.axis_index("s") * NC + lax.axis_index("c")
    base = wid * PER_W

    pltpu.sync_copy(edge_hbm.at[0, pl.ds(base, PER_W)], sidx)
    pltpu.sync_copy(edge_hbm.at[1, pl.ds(base, PER_W)], didx)

    bufs = ((arows0, brows0, sem0), (arows1, brows1, sem1))
    himask = jnp.full((L,), -65536, jnp.int32)

    def start(c, p):
        ar, br, sem = bufs[p]
        off = c * CHUNK
        pltpu.async_copy(a_hbm.at[sidx.at[pl.ds(off, CHUNK)]], ar, sem)
        pltpu.async_copy(b_hbm.at[didx.at[pl.ds(off, CHUNK)]], br, sem)

    def finish(c, p):
        ar, br, sem = bufs[p]
        pltpu.make_async_copy(a_hbm.at[pl.ds(0, CHUNK)], ar, sem).wait()
        pltpu.make_async_copy(b_hbm.at[pl.ds(0, CHUNK)], br, sem).wait()
        lane = lax.iota(jnp.int32, L)
        for g in range(CHUNK // L):
            rows = jnp.full((L,), g * L, jnp.int32) + lane

            def kbody(kk, carry):
                col, acc_e, acc_o = carry
                for _ in range(KU):
                    va = plsc.load_gather(ar, [rows, col])
                    vb = plsc.load_gather(br, [rows, col])
                    ae = plsc.bitcast(lax.shift_left(va, 16), jnp.float32)
                    be = plsc.bitcast(lax.shift_left(vb, 16), jnp.float32)
                    ao = plsc.bitcast(va & himask, jnp.float32)
                    bo = plsc.bitcast(vb & himask, jnp.float32)
                    acc_e = acc_e + ae * be
                    acc_o = acc_o + ao * bo
                    col = (col + 1) & (DP - 1)
                return col, acc_e, acc_o

            _, acc_e, acc_o = lax.fori_loop(
                0, DP // KU, kbody,
                (lane, jnp.zeros((L,), jnp.float32),
                 jnp.zeros((L,), jnp.float32)))
            z[pl.ds(c * CHUNK + g * L, L)] = acc_e + acc_o

    start(0, 0)

    def pair_body(i, carry):
        c = 2 * i
        start(c + 1, 1)
        finish(c, 0)
        start(c + 2, 0)
        finish(c + 1, 1)
        return carry

    lax.fori_loop(0, (N_FULL - 2) // 2, pair_body, 0)
    start(N_FULL - 1, 1)
    finish(N_FULL - 2, 0)

    art = arows0.at[pl.ds(0, TAIL)]
    brt = brows0.at[pl.ds(0, TAIL)]
    pltpu.async_copy(a_hbm.at[sidx.at[pl.ds(TAIL_OFF, TAIL)]], art, sem0)
    pltpu.async_copy(b_hbm.at[didx.at[pl.ds(TAIL_OFF, TAIL)]], brt, sem0)
    finish(N_FULL - 1, 1)
    pltpu.make_async_copy(a_hbm.at[pl.ds(0, TAIL)], art, sem0).wait()
    pltpu.make_async_copy(b_hbm.at[pl.ds(0, TAIL)], brt, sem0).wait()
    lane_t = lax.iota(jnp.int32, L)

    def tbody(kk, carry):
        col, acc_e, acc_o = carry
        for _ in range(KU):
            va = plsc.load_gather(arows0, [lane_t, col])
            vb = plsc.load_gather(brows0, [lane_t, col])
            ae = plsc.bitcast(lax.shift_left(va, 16), jnp.float32)
            be = plsc.bitcast(lax.shift_left(vb, 16), jnp.float32)
            ao = plsc.bitcast(va & himask, jnp.float32)
            bo = plsc.bitcast(vb & himask, jnp.float32)
            acc_e = acc_e + ae * be
            acc_o = acc_o + ao * bo
            col = (col + 1) & (DP - 1)
        return col, acc_e, acc_o

    _, acc_e, acc_o = lax.fori_loop(
        0, DP // KU, tbody,
        (lane_t, jnp.zeros((L,), jnp.float32), jnp.zeros((L,), jnp.float32)))
    z[pl.ds(TAIL_OFF, L)] = acc_e + acc_o
    pltpu.sync_copy(z, out_hbm.at[pl.ds(base, PER_W)])


@jax.jit
def _sc_edge_dot(a, b, edge):
    f = functools.partial(
        pl.kernel,
        mesh=plsc.VectorSubcoreMesh(**_MESH),
        compiler_params=_SC_PARAMS,
        out_type=jax.ShapeDtypeStruct((N_EDGES,), jnp.float32),
        scratch_types=[
            pltpu.VMEM((PER_W,), jnp.int32),
            pltpu.VMEM((PER_W,), jnp.int32),
            pltpu.VMEM((CHUNK, DP), jnp.int32),
            pltpu.VMEM((CHUNK, DP), jnp.int32),
            pltpu.VMEM((CHUNK, DP), jnp.int32),
            pltpu.VMEM((CHUNK, DP), jnp.int32),
            pltpu.VMEM((PER_W,), jnp.float32),
            pltpu.SemaphoreType.DMA,
            pltpu.SemaphoreType.DMA,
        ],
    )(_edge_dot_body)
    return f(a, b, edge)


def kernel(emb, edge_index, d, scale):
    ds = (d * scale).astype(jnp.float32)
    a_p, b_p = _sc_prep(emb, ds)
    z = _sc_edge_dot(a_p, b_p, edge_index.astype(jnp.int32))
    return z.reshape(N_EDGES, 1)
